# baseline (speedup 1.0000x reference)
import functools

import jax
import jax.numpy as jnp
from jax import lax
from jax.experimental import pallas as pl
from jax.experimental.pallas import tpu as pltpu
from jax.experimental.pallas import tpu_sc as plsc

N = 10000
E = 320000
D = 128
C = 128

NC = 2
NS = 16
NW = NC * NS
EP = E // NW
CH = 128
NCH = EP // CH
MAIN = NCH * CH
TAIL = EP - MAIN
GRP = CH // 16
NPAD = 10240
RP = NPAD // NS

ROW_BLK = 1000
SLOPE = 0.2


def _proj_body(x_ref, w_ref, a2_ref, xw_ref, s12_ref):
    xw = jnp.dot(x_ref[...], w_ref[...], preferred_element_type=jnp.float32)
    xw_ref[...] = xw
    s12_ref[...] = jnp.dot(xw, a2_ref[...], preferred_element_type=jnp.float32)


def _project(x, weight, a2):
    return pl.pallas_call(
        _proj_body,
        grid=(N // ROW_BLK,),
        in_specs=[
            pl.BlockSpec((ROW_BLK, D), lambda i: (i, 0)),
            pl.BlockSpec((D, C), lambda i: (0, 0)),
            pl.BlockSpec((C, 2), lambda i: (0, 0)),
        ],
        out_specs=[
            pl.BlockSpec((ROW_BLK, C), lambda i: (i, 0)),
            pl.BlockSpec((ROW_BLK, 2), lambda i: (i, 0)),
        ],
        out_shape=[
            jax.ShapeDtypeStruct((N, C), jnp.float32),
            jax.ShapeDtypeStruct((N, 2), jnp.float32),
        ],
    )(x, weight, a2)


NCHT = NCH + 1
M32 = -65536


def _sc_body(xw_hbm, sv_hbm, pack_hbm, z_hbm,
             part_hbm, am_hbm, at_hbm,
             acc, sv_v, pack_v, alpha_db, rows_v, asc,
             gsem, ssem, psem, asem):
    cid = lax.axis_index("c")
    sid = lax.axis_index("s")
    wid = cid * NS + sid

    pltpu.sync_copy(sv_hbm, sv_v)
    pltpu.sync_copy(z_hbm, acc.at[pl.ds(sid * RP, RP)])
    plsc.subcore_barrier()

    def compute_group(g, b):
        rix = pack_v[3 * b, pl.ds(g * 16, 16)]
        cix = pack_v[3 * b + 1, pl.ds(g * 16, 16)]
        eav = plsc.bitcast(pack_v[3 * b + 2, pl.ds(g * 16, 16)], jnp.float32)
        wr = plsc.load_gather(sv_v, [rix])
        wc = plsc.load_gather(sv_v, [cix])
        s1r = plsc.bitcast(jnp.bitwise_and(wr, jnp.int32(M32)), jnp.float32)
        s2c = plsc.bitcast(jnp.left_shift(wc, 16), jnp.float32)
        t = s1r + s2c
        al = jnp.maximum(t, t * SLOPE) * eav
        alpha_db[b, pl.ds(g * 16, 16)] = al
        asc[pl.ds(0, 16)] = al
        asc[pl.ds(16, 16)] = al
        for e in range(16):
            sp = plsc.load_gather(asc, [jnp.full((16,), 16 + e, jnp.int32)])
            for f in range(C // 16):
                sl = pl.ds(f * 16, 16)
                rows_v[b, g * 16 + e, sl] = rows_v[b, g * 16 + e, sl] * sp
        return b

    def chunk(c, carry):
        b = jnp.bitwise_and(c, 1)
        nb = 1 - b

        @pl.when(c >= 1)
        def _():
            pltpu.make_async_copy(rows_v.at[0], acc.at[pl.ds(0, CH)],
                                  ssem).wait()

        @pl.when(c + 1 < NCHT)
        def _():
            pltpu.async_copy(pack_hbm.at[wid, c + 1],
                             pack_v.at[pl.ds(3 * nb, 3)], psem)

        pltpu.make_async_copy(xw_hbm.at[pl.ds(0, CH)], rows_v.at[0],
                              gsem).wait()

        @pl.when(c >= 1)
        def _():
            pltpu.make_async_copy(alpha_db.at[0],
                                  am_hbm.at[pl.ds(0, CH)], asem).wait()

        lax.fori_loop(0, GRP, compute_group, b)

        @pl.when(c < NCH)
        def _():
            pltpu.async_copy(alpha_db.at[b],
                             am_hbm.at[pl.ds(wid * MAIN + c * CH, CH)], asem)

        @pl.when(c == NCH)
        def _():
            pltpu.sync_copy(alpha_db.at[b, pl.ds(0, TAIL)],
                            at_hbm.at[pl.ds(wid * TAIL, TAIL)])

        pltpu.async_copy(rows_v.at[b], acc.at[pack_v.at[3 * b]], ssem,
                         add=True)

        @pl.when(c + 1 < NCHT)
        def _():
            pltpu.make_async_copy(pack_hbm.at[wid, 0],
                                  pack_v.at[pl.ds(0, 3)], psem).wait()
            pltpu.async_copy(xw_hbm.at[pack_v.at[3 * nb + 1]],
                             rows_v.at[nb], gsem)

        return carry

    pltpu.sync_copy(pack_hbm.at[wid, 0], pack_v.at[pl.ds(0, 3)])
    pltpu.async_copy(xw_hbm.at[pack_v.at[1]], rows_v.at[0], gsem)
    lax.fori_loop(0, NCHT, chunk, 0)
    pltpu.make_async_copy(rows_v.at[0], acc.at[pl.ds(0, CH)], ssem).wait()

    plsc.subcore_barrier()

    pltpu.sync_copy(acc.at[pl.ds(sid * RP, RP)],
                    part_hbm.at[pl.ds(cid * NPAD + sid * RP, RP)])


def _sc_edges(xw, sv, pack):
    mesh = plsc.VectorSubcoreMesh(core_axis_name="c", subcore_axis_name="s",
                                  num_cores=NC, num_subcores=NS)
    zrows = jnp.zeros((RP, C), jnp.float32)
    f = pl.kernel(
        _sc_body,
        out_type=[
            jax.ShapeDtypeStruct((NC * NPAD, C), jnp.float32),
            jax.ShapeDtypeStruct((NW * MAIN,), jnp.float32),
            jax.ShapeDtypeStruct((NW * TAIL,), jnp.float32),
        ],
        mesh=mesh,
        scratch_types=[
            pltpu.VMEM_SHARED((NPAD, C), jnp.float32),
            pltpu.VMEM((N,), jnp.int32),
            pltpu.VMEM((6, CH), jnp.int32),
            pltpu.VMEM((2, CH), jnp.float32),
            pltpu.VMEM((2, CH, C), jnp.float32),
            pltpu.VMEM((32,), jnp.float32),
            pltpu.SemaphoreType.DMA,
            pltpu.SemaphoreType.DMA,
            pltpu.SemaphoreType.DMA,
            pltpu.SemaphoreType.DMA,
        ],
        compiler_params=pltpu.CompilerParams(needs_layout_passes=False),
    )
    return f(xw, sv, pack, zrows)


def _final_body(p0_ref, p1_ref, xw_ref, s12_ref, b_ref, out_ref, al_ref):
    t = s12_ref[:, 0:1] + s12_ref[:, 1:2]
    al = jnp.maximum(t, t * SLOPE)
    al_ref[...] = al
    out_ref[...] = (p0_ref[...] + p1_ref[...] + al * xw_ref[...] + b_ref[...])


def _finalize(p0, p1, xw, s12, bias2d):
    return pl.pallas_call(
        _final_body,
        grid=(N // ROW_BLK,),
        in_specs=[
            pl.BlockSpec((ROW_BLK, C), lambda i: (i, 0)),
            pl.BlockSpec((ROW_BLK, C), lambda i: (i, 0)),
            pl.BlockSpec((ROW_BLK, C), lambda i: (i, 0)),
            pl.BlockSpec((ROW_BLK, 2), lambda i: (i, 0)),
            pl.BlockSpec((1, C), lambda i: (0, 0)),
        ],
        out_specs=[
            pl.BlockSpec((ROW_BLK, C), lambda i: (i, 0)),
            pl.BlockSpec((ROW_BLK, 1), lambda i: (i, 0)),
        ],
        out_shape=[
            jax.ShapeDtypeStruct((N, C), jnp.float32),
            jax.ShapeDtypeStruct((N, 1), jnp.float32),
        ],
    )(p0, p1, xw, s12, bias2d)


def kernel(x, edge_index, edge_attr, weight, att_weight, bias):
    aw = att_weight.reshape(2 * C)
    a2 = jnp.stack([aw[:C], aw[C:]], axis=1)

    xw, s12 = _project(x, weight, a2)
    s1 = s12[:, 0]
    s2 = s12[:, 1]

    row32 = edge_index[0].astype(jnp.int32).reshape(NW, EP)
    col32 = edge_index[1].astype(jnp.int32).reshape(NW, EP)
    eabits = lax.bitcast_convert_type(
        edge_attr.reshape(-1).astype(jnp.float32), jnp.int32
    ).reshape(NW, EP)

    s1b = lax.bitcast_convert_type(s1, jnp.int32)
    s2b = lax.bitcast_convert_type(s2, jnp.int32)
    hi = jnp.bitwise_and(s1b + 32768, jnp.int32(-65536))
    lo = jnp.bitwise_and(jnp.right_shift(s2b + 32768, 16), jnp.int32(65535))
    sv = jnp.bitwise_or(hi, lo)

    dump = jnp.broadcast_to(N + jnp.arange(CH - TAIL, dtype=jnp.int32),
                            (NW, CH - TAIL))
    zpad = jnp.zeros((NW, CH - TAIL), jnp.int32)

    def _chunked(a, pad):
        main = a[:, :MAIN].reshape(NW, NCH, CH)
        tail = jnp.concatenate([a[:, MAIN:], pad], axis=1)[:, None, :]
        return jnp.concatenate([main, tail], axis=1)

    pack = jnp.stack(
        [_chunked(row32, dump), _chunked(col32, zpad), _chunked(eabits, zpad)],
        axis=2)

    part, am, at_ = _sc_edges(xw, sv, pack)
    alpha_e = jnp.concatenate(
        [am.reshape(NW, MAIN), at_.reshape(NW, TAIL)], axis=1).reshape(E)

    out, alpha_loop = _finalize(part[:N], part[NPAD:NPAD + N], xw, s12,
                                bias.reshape(1, C))

    loop = jnp.arange(N, dtype=edge_index.dtype)
    edge_index_out = jnp.concatenate(
        [edge_index, jnp.stack([loop, loop])], axis=1)
    alpha = jnp.concatenate([alpha_e[:, None], alpha_loop], axis=0)
    return out, edge_index_out, alpha


# speedup vs baseline: 1.5816x; 1.5816x over previous
import functools

import jax
import jax.numpy as jnp
from jax import lax
from jax.experimental import pallas as pl
from jax.experimental.pallas import tpu as pltpu
from jax.experimental.pallas import tpu_sc as plsc

N = 10000
E = 320000
D = 128
C = 128

NC = 2
NS = 16
NW = NC * NS
EP = E // NW
CH = 128
NCH = EP // CH
MAIN = NCH * CH
TAIL = EP - MAIN
GRP = CH // 16
NPAD = 10240
RP = NPAD // NS

ROW_BLK = 1000
SLOPE = 0.2


def _proj_body(x_ref, w_ref, a2_ref, xw_ref, s12_ref):
    xw = jnp.dot(x_ref[...], w_ref[...], preferred_element_type=jnp.float32)
    xw_ref[...] = xw
    s12_ref[...] = jnp.dot(xw, a2_ref[...], preferred_element_type=jnp.float32)


def _project(x, weight, a2):
    return pl.pallas_call(
        _proj_body,
        grid=(N // ROW_BLK,),
        in_specs=[
            pl.BlockSpec((ROW_BLK, D), lambda i: (i, 0)),
            pl.BlockSpec((D, C), lambda i: (0, 0)),
            pl.BlockSpec((C, 2), lambda i: (0, 0)),
        ],
        out_specs=[
            pl.BlockSpec((ROW_BLK, C), lambda i: (i, 0)),
            pl.BlockSpec((ROW_BLK, 2), lambda i: (i, 0)),
        ],
        out_shape=[
            jax.ShapeDtypeStruct((N, C), jnp.float32),
            jax.ShapeDtypeStruct((N, 2), jnp.float32),
        ],
    )(x, weight, a2)


NCHT = NCH + 1
M32 = -65536


def _sc_body(xw_hbm, sv_hbm, pack_hbm, z_hbm,
             part_hbm, am_hbm, at_hbm,
             acc, sv_v, pack_v, alpha_db, rows_v, asc,
             gsem, ssem, psem, asem):
    cid = lax.axis_index("c")
    sid = lax.axis_index("s")
    wid = cid * NS + sid

    pltpu.sync_copy(sv_hbm, sv_v)
    pltpu.sync_copy(z_hbm, acc.at[pl.ds(sid * RP, RP)])
    plsc.subcore_barrier()

    def compute_group(b, p):
        def body(g, dep):
            rix = pack_v[8 * p, pl.ds(g * 16, 16)]
            cix = pack_v[8 * p + 1, pl.ds(g * 16, 16)]
            eav = plsc.bitcast(pack_v[8 * p + 2, pl.ds(g * 16, 16)],
                               jnp.float32)
            wr = plsc.load_gather(sv_v, [rix])
            wc = plsc.load_gather(sv_v, [cix])
            s1r = plsc.bitcast(jnp.bitwise_and(wr, jnp.int32(M32)),
                               jnp.float32)
            s2c = plsc.bitcast(jnp.left_shift(wc, 16), jnp.float32)
            t = s1r + s2c
            al = jnp.maximum(t, t * SLOPE) * eav
            alpha_db[b, pl.ds(g * 16, 16)] = al
            asc[pl.ds(0, 16)] = al
            asc[pl.ds(16, 16)] = al
            for e in range(16):
                sp = plsc.load_gather(asc,
                                      [jnp.full((16,), 16 + e, jnp.int32)])
                for f in range(C // 16):
                    sl = pl.ds(f * 16, 16)
                    rows_v[b, g * 16 + e, sl] = rows_v[b, g * 16 + e, sl] * sp
            return dep + jnp.sum(al)
        return body

    def step(c, b, p, first=False, has_next=True, has_next2=True,
             tail=False):
        pn = (p + 1) % 4
        pi = (p + 2) % 4
        nb = 1 - b
        if has_next:
            pltpu.make_async_copy(pack_hbm.at[wid, 0],
                                  pack_v.at[pl.ds(0, 3)], psem).wait()
        pltpu.make_async_copy(xw_hbm.at[pl.ds(0, CH)], rows_v.at[0],
                              gsem).wait()
        if not first:
            pltpu.make_async_copy(alpha_db.at[0], am_hbm.at[pl.ds(0, CH)],
                                  asem).wait()
        dep = lax.fori_loop(0, GRP, compute_group(b, p), jnp.float32(0.0))
        if not first:
            pltpu.make_async_copy(rows_v.at[0], acc.at[pl.ds(0, CH)],
                                  ssem).wait()

        @pl.when(dep < jnp.float32(3e38))
        def _():
            if tail:
                pltpu.sync_copy(alpha_db.at[b, pl.ds(0, TAIL)],
                                at_hbm.at[pl.ds(wid * TAIL, TAIL)])
            else:
                pltpu.async_copy(
                    alpha_db.at[b],
                    am_hbm.at[pl.ds(wid * MAIN + c * CH, CH)], asem)
            if has_next:
                pltpu.async_copy(xw_hbm.at[pack_v.at[8 * pn + 1]],
                                 rows_v.at[nb], gsem)
            if has_next2:
                pltpu.async_copy(pack_hbm.at[wid, c + 2],
                                 pack_v.at[pl.ds(8 * pi, 3)], psem)
            pltpu.async_copy(rows_v.at[b], acc.at[pack_v.at[8 * p]], ssem,
                             add=True)

    pltpu.sync_copy(pack_hbm.at[wid, 0], pack_v.at[pl.ds(0, 3)])
    pltpu.async_copy(xw_hbm.at[pack_v.at[1]], rows_v.at[0], gsem)
    pltpu.async_copy(pack_hbm.at[wid, 1], pack_v.at[pl.ds(8, 3)], psem)

    step(0, 0, 0, first=True)

    def quad(i, carry):
        c0 = 4 * i + 1
        step(c0, 1, 1)
        step(c0 + 1, 0, 2)
        step(c0 + 2, 1, 3)
        step(c0 + 3, 0, 0)
        return carry

    lax.fori_loop(0, (NCHT - 7) // 4, quad, 0)
    base = 1 + 4 * ((NCHT - 7) // 4)
    step(base, 1, 1)
    step(base + 1, 0, 2)
    step(base + 2, 1, 3)
    step(base + 3, 0, 0)
    step(base + 4, 1, 1, has_next2=False)
    step(base + 5, 0, 2, has_next=False, has_next2=False, tail=True)

    pltpu.make_async_copy(rows_v.at[0], acc.at[pl.ds(0, CH)], ssem).wait()

    plsc.subcore_barrier()

    pltpu.sync_copy(acc.at[pl.ds(sid * RP, RP)],
                    part_hbm.at[pl.ds(cid * NPAD + sid * RP, RP)])


def _sc_edges(xw, sv, pack):
    mesh = plsc.VectorSubcoreMesh(core_axis_name="c", subcore_axis_name="s",
                                  num_cores=NC, num_subcores=NS)
    zrows = jnp.zeros((RP, C), jnp.float32)
    f = pl.kernel(
        _sc_body,
        out_type=[
            jax.ShapeDtypeStruct((NC * NPAD, C), jnp.float32),
            jax.ShapeDtypeStruct((NW * MAIN,), jnp.float32),
            jax.ShapeDtypeStruct((NW * TAIL,), jnp.float32),
        ],
        mesh=mesh,
        scratch_types=[
            pltpu.VMEM_SHARED((NPAD, C), jnp.float32),
            pltpu.VMEM((N,), jnp.int32),
            pltpu.VMEM((32, CH), jnp.int32),
            pltpu.VMEM((2, CH), jnp.float32),
            pltpu.VMEM((2, CH, C), jnp.float32),
            pltpu.VMEM((32,), jnp.float32),
            pltpu.SemaphoreType.DMA,
            pltpu.SemaphoreType.DMA,
            pltpu.SemaphoreType.DMA,
            pltpu.SemaphoreType.DMA,
        ],
        compiler_params=pltpu.CompilerParams(needs_layout_passes=False),
    )
    return f(xw, sv, pack, zrows)


def _final_body(p0_ref, p1_ref, xw_ref, s12_ref, b_ref, out_ref, al_ref):
    t = s12_ref[:, 0:1] + s12_ref[:, 1:2]
    al = jnp.maximum(t, t * SLOPE)
    al_ref[...] = al
    out_ref[...] = (p0_ref[...] + p1_ref[...] + al * xw_ref[...] + b_ref[...])


def _finalize(p0, p1, xw, s12, bias2d):
    return pl.pallas_call(
        _final_body,
        grid=(N // ROW_BLK,),
        in_specs=[
            pl.BlockSpec((ROW_BLK, C), lambda i: (i, 0)),
            pl.BlockSpec((ROW_BLK, C), lambda i: (i, 0)),
            pl.BlockSpec((ROW_BLK, C), lambda i: (i, 0)),
            pl.BlockSpec((ROW_BLK, 2), lambda i: (i, 0)),
            pl.BlockSpec((1, C), lambda i: (0, 0)),
        ],
        out_specs=[
            pl.BlockSpec((ROW_BLK, C), lambda i: (i, 0)),
            pl.BlockSpec((ROW_BLK, 1), lambda i: (i, 0)),
        ],
        out_shape=[
            jax.ShapeDtypeStruct((N, C), jnp.float32),
            jax.ShapeDtypeStruct((N, 1), jnp.float32),
        ],
    )(p0, p1, xw, s12, bias2d)


def kernel(x, edge_index, edge_attr, weight, att_weight, bias):
    aw = att_weight.reshape(2 * C)
    a2 = jnp.stack([aw[:C], aw[C:]], axis=1)

    xw, s12 = _project(x, weight, a2)
    s1 = s12[:, 0]
    s2 = s12[:, 1]

    row32 = edge_index[0].astype(jnp.int32).reshape(NW, EP)
    col32 = edge_index[1].astype(jnp.int32).reshape(NW, EP)
    eabits = lax.bitcast_convert_type(
        edge_attr.reshape(-1).astype(jnp.float32), jnp.int32
    ).reshape(NW, EP)

    s1b = lax.bitcast_convert_type(s1, jnp.int32)
    s2b = lax.bitcast_convert_type(s2, jnp.int32)
    hi = jnp.bitwise_and(s1b + 32768, jnp.int32(-65536))
    lo = jnp.bitwise_and(jnp.right_shift(s2b + 32768, 16), jnp.int32(65535))
    sv = jnp.bitwise_or(hi, lo)

    dump = jnp.broadcast_to(N + jnp.arange(CH - TAIL, dtype=jnp.int32),
                            (NW, CH - TAIL))
    zpad = jnp.zeros((NW, CH - TAIL), jnp.int32)

    def _chunked(a, pad):
        main = a[:, :MAIN].reshape(NW, NCH, CH)
        tail = jnp.concatenate([a[:, MAIN:], pad], axis=1)[:, None, :]
        return jnp.concatenate([main, tail], axis=1)

    pack = jnp.stack(
        [_chunked(row32, dump), _chunked(col32, zpad), _chunked(eabits, zpad)],
        axis=2)

    part, am, at_ = _sc_edges(xw, sv, pack)
    alpha_e = jnp.concatenate(
        [am.reshape(NW, MAIN), at_.reshape(NW, TAIL)], axis=1).reshape(E)

    out, alpha_loop = _finalize(part[:N], part[NPAD:NPAD + N], xw, s12,
                                bias.reshape(1, C))

    loop = jnp.arange(N, dtype=edge_index.dtype)
    edge_index_out = jnp.concatenate(
        [edge_index, jnp.stack([loop, loop])], axis=1)
    alpha = jnp.concatenate([alpha_e[:, None], alpha_loop], axis=0)
    return out, edge_index_out, alpha


# speedup vs baseline: 1.7889x; 1.1311x over previous
import functools

import jax
import jax.numpy as jnp
from jax import lax
from jax.experimental import pallas as pl
from jax.experimental.pallas import tpu as pltpu
from jax.experimental.pallas import tpu_sc as plsc

N = 10000
E = 320000
D = 128
C = 128

NC = 2
NS = 16
NW = NC * NS
EP = E // NW
CH = 128
NCH = EP // CH
MAIN = NCH * CH
TAIL = EP - MAIN
GRP = CH // 16
NPAD = 10240
RP = NPAD // NS

ROW_BLK = 1000
SLOPE = 0.2


def _proj_body(x_ref, w_ref, a2_ref, xw_ref, s12_ref):
    xw = jnp.dot(x_ref[...], w_ref[...], preferred_element_type=jnp.float32)
    xw_ref[...] = xw
    s12_ref[...] = jnp.dot(xw, a2_ref[...], preferred_element_type=jnp.float32)


def _project(x, weight, a2):
    return pl.pallas_call(
        _proj_body,
        grid=(N // ROW_BLK,),
        in_specs=[
            pl.BlockSpec((ROW_BLK, D), lambda i: (i, 0)),
            pl.BlockSpec((D, C), lambda i: (0, 0)),
            pl.BlockSpec((C, 2), lambda i: (0, 0)),
        ],
        out_specs=[
            pl.BlockSpec((ROW_BLK, C), lambda i: (i, 0)),
            pl.BlockSpec((ROW_BLK, 2), lambda i: (i, 0)),
        ],
        out_shape=[
            jax.ShapeDtypeStruct((N, C), jnp.float32),
            jax.ShapeDtypeStruct((N, 2), jnp.float32),
        ],
    )(x, weight, a2)


NCHT = NCH + 1
M32 = -65536


def _sc_body(xw_hbm, sv_hbm, pack_hbm, z_hbm,
             part_hbm, am_hbm, at_hbm,
             acc, sv_v, pack_v, alpha_db, rows_v, asc,
             gsem, ssem, psem, asem):
    cid = lax.axis_index("c")
    sid = lax.axis_index("s")
    wid = cid * NS + sid

    pltpu.sync_copy(sv_hbm, sv_v)
    pltpu.sync_copy(z_hbm, acc.at[pl.ds(sid * RP, RP)])
    plsc.subcore_barrier()

    def compute_group(b, p):
        def body(g, dep):
            rix = pack_v[8 * p, pl.ds(g * 16, 16)]
            cix = pack_v[8 * p + 1, pl.ds(g * 16, 16)]
            eav = plsc.bitcast(pack_v[8 * p + 2, pl.ds(g * 16, 16)],
                               jnp.float32)
            wr = plsc.load_gather(sv_v, [rix])
            wc = plsc.load_gather(sv_v, [cix])
            s1r = plsc.bitcast(jnp.bitwise_and(wr, jnp.int32(M32)),
                               jnp.float32)
            s2c = plsc.bitcast(jnp.left_shift(wc, 16), jnp.float32)
            t = s1r + s2c
            al = jnp.maximum(t, t * SLOPE) * eav
            alpha_db[b, pl.ds(g * 16, 16)] = al
            asc[pl.ds(0, 16)] = al
            asc[pl.ds(16, 16)] = al
            for e in range(16):
                sp = plsc.load_gather(asc,
                                      [jnp.full((16,), 16 + e, jnp.int32)])
                for f in range(C // 16):
                    sl = pl.ds(f * 16, 16)
                    rows_v[b, g * 16 + e, sl] = rows_v[b, g * 16 + e, sl] * sp
            return dep + jnp.sum(al)
        return body

    def step(c, b, p, first=False, has_next=True, has_next2=True,
             tail=False):
        pn = (p + 1) % 4
        pi = (p + 2) % 4
        nb = 1 - b
        if has_next:
            pltpu.make_async_copy(pack_hbm.at[wid, 0],
                                  pack_v.at[pl.ds(0, 3)], psem).wait()
        if not first:
            @pl.when(c >= 2)
            def _():
                pltpu.make_async_copy(alpha_db.at[0],
                                      am_hbm.at[pl.ds(0, CH)], asem).wait()
            pltpu.make_async_copy(rows_v.at[0], acc.at[pl.ds(0, CH)],
                                  ssem).wait()
        if has_next:
            pltpu.async_copy(xw_hbm.at[pack_v.at[8 * pn + 1]],
                             rows_v.at[nb], gsem)
        if has_next2:
            pltpu.async_copy(pack_hbm.at[wid, c + 2],
                             pack_v.at[pl.ds(8 * pi, 3)], psem)
        pltpu.make_async_copy(xw_hbm.at[pl.ds(0, CH)], rows_v.at[0],
                              gsem).wait()
        dep = lax.fori_loop(0, GRP, compute_group(b, p), jnp.float32(0.0))

        @pl.when(dep < jnp.float32(3e38))
        def _():
            if tail:
                pltpu.sync_copy(alpha_db.at[b, pl.ds(0, TAIL)],
                                at_hbm.at[pl.ds(wid * TAIL, TAIL)])
            else:
                pltpu.async_copy(
                    alpha_db.at[b],
                    am_hbm.at[pl.ds(wid * MAIN + c * CH, CH)], asem)
            pltpu.async_copy(rows_v.at[b], acc.at[pack_v.at[8 * p]], ssem,
                             add=True)

    pltpu.sync_copy(pack_hbm.at[wid, 0], pack_v.at[pl.ds(0, 3)])
    pltpu.async_copy(xw_hbm.at[pack_v.at[1]], rows_v.at[0], gsem)
    pltpu.async_copy(pack_hbm.at[wid, 1], pack_v.at[pl.ds(8, 3)], psem)

    step(0, 0, 0, first=True)

    def quad(i, carry):
        c0 = 4 * i + 1
        step(c0, 1, 1)
        step(c0 + 1, 0, 2)
        step(c0 + 2, 1, 3)
        step(c0 + 3, 0, 0)
        return carry

    lax.fori_loop(0, (NCHT - 7) // 4, quad, 0)
    base = 1 + 4 * ((NCHT - 7) // 4)
    step(base, 1, 1)
    step(base + 1, 0, 2)
    step(base + 2, 1, 3)
    step(base + 3, 0, 0)
    step(base + 4, 1, 1, has_next2=False)
    step(base + 5, 0, 2, has_next=False, has_next2=False, tail=True)

    pltpu.make_async_copy(rows_v.at[0], acc.at[pl.ds(0, CH)], ssem).wait()
    pltpu.make_async_copy(alpha_db.at[0], am_hbm.at[pl.ds(0, CH)],
                          asem).wait()

    plsc.subcore_barrier()

    pltpu.sync_copy(acc.at[pl.ds(sid * RP, RP)],
                    part_hbm.at[pl.ds(cid * NPAD + sid * RP, RP)])


def _sc_edges(xw, sv, pack):
    mesh = plsc.VectorSubcoreMesh(core_axis_name="c", subcore_axis_name="s",
                                  num_cores=NC, num_subcores=NS)
    zrows = jnp.zeros((RP, C), jnp.float32)
    f = pl.kernel(
        _sc_body,
        out_type=[
            jax.ShapeDtypeStruct((NC * NPAD, C), jnp.float32),
            jax.ShapeDtypeStruct((NW * MAIN,), jnp.float32),
            jax.ShapeDtypeStruct((NW * TAIL,), jnp.float32),
        ],
        mesh=mesh,
        scratch_types=[
            pltpu.VMEM_SHARED((NPAD, C), jnp.float32),
            pltpu.VMEM((N,), jnp.int32),
            pltpu.VMEM((32, CH), jnp.int32),
            pltpu.VMEM((2, CH), jnp.float32),
            pltpu.VMEM((2, CH, C), jnp.float32),
            pltpu.VMEM((32,), jnp.float32),
            pltpu.SemaphoreType.DMA,
            pltpu.SemaphoreType.DMA,
            pltpu.SemaphoreType.DMA,
            pltpu.SemaphoreType.DMA,
        ],
        compiler_params=pltpu.CompilerParams(needs_layout_passes=False),
    )
    return f(xw, sv, pack, zrows)


def _final_body(p0_ref, p1_ref, xw_ref, s12_ref, b_ref, out_ref, al_ref):
    t = s12_ref[:, 0:1] + s12_ref[:, 1:2]
    al = jnp.maximum(t, t * SLOPE)
    al_ref[...] = al
    out_ref[...] = (p0_ref[...] + p1_ref[...] + al * xw_ref[...] + b_ref[...])


def _finalize(p0, p1, xw, s12, bias2d):
    return pl.pallas_call(
        _final_body,
        grid=(N // ROW_BLK,),
        in_specs=[
            pl.BlockSpec((ROW_BLK, C), lambda i: (i, 0)),
            pl.BlockSpec((ROW_BLK, C), lambda i: (i, 0)),
            pl.BlockSpec((ROW_BLK, C), lambda i: (i, 0)),
            pl.BlockSpec((ROW_BLK, 2), lambda i: (i, 0)),
            pl.BlockSpec((1, C), lambda i: (0, 0)),
        ],
        out_specs=[
            pl.BlockSpec((ROW_BLK, C), lambda i: (i, 0)),
            pl.BlockSpec((ROW_BLK, 1), lambda i: (i, 0)),
        ],
        out_shape=[
            jax.ShapeDtypeStruct((N, C), jnp.float32),
            jax.ShapeDtypeStruct((N, 1), jnp.float32),
        ],
    )(p0, p1, xw, s12, bias2d)


def kernel(x, edge_index, edge_attr, weight, att_weight, bias):
    aw = att_weight.reshape(2 * C)
    a2 = jnp.stack([aw[:C], aw[C:]], axis=1)

    xw, s12 = _project(x, weight, a2)
    s1 = s12[:, 0]
    s2 = s12[:, 1]

    row32 = edge_index[0].astype(jnp.int32).reshape(NW, EP)
    col32 = edge_index[1].astype(jnp.int32).reshape(NW, EP)
    eabits = lax.bitcast_convert_type(
        edge_attr.reshape(-1).astype(jnp.float32), jnp.int32
    ).reshape(NW, EP)

    s1b = lax.bitcast_convert_type(s1, jnp.int32)
    s2b = lax.bitcast_convert_type(s2, jnp.int32)
    hi = jnp.bitwise_and(s1b + 32768, jnp.int32(-65536))
    lo = jnp.bitwise_and(jnp.right_shift(s2b + 32768, 16), jnp.int32(65535))
    sv = jnp.bitwise_or(hi, lo)

    dump = jnp.broadcast_to(N + jnp.arange(CH - TAIL, dtype=jnp.int32),
                            (NW, CH - TAIL))
    zpad = jnp.zeros((NW, CH - TAIL), jnp.int32)

    def _chunked(a, pad):
        main = a[:, :MAIN].reshape(NW, NCH, CH)
        tail = jnp.concatenate([a[:, MAIN:], pad], axis=1)[:, None, :]
        return jnp.concatenate([main, tail], axis=1)

    pack = jnp.stack(
        [_chunked(row32, dump), _chunked(col32, zpad), _chunked(eabits, zpad)],
        axis=2)

    part, am, at_ = _sc_edges(xw, sv, pack)
    alpha_e = jnp.concatenate(
        [am.reshape(NW, MAIN), at_.reshape(NW, TAIL)], axis=1).reshape(E)

    out, alpha_loop = _finalize(part[:N], part[NPAD:NPAD + N], xw, s12,
                                bias.reshape(1, C))

    loop = jnp.arange(N, dtype=edge_index.dtype)
    edge_index_out = jnp.concatenate(
        [edge_index, jnp.stack([loop, loop])], axis=1)
    alpha = jnp.concatenate([alpha_e[:, None], alpha_loop], axis=0)
    return out, edge_index_out, alpha
